# baseline (speedup 1.0000x reference)
import jax
import jax.numpy as jnp
from jax import lax
from jax.experimental import pallas as pl
from jax.experimental.pallas import tpu as pltpu
from jax.experimental.pallas import tpu_sc as plsc

BATCH = 16384
FIELDS = 100
EMBED_DIM = 128
VOCAB = 100000

NC = 2
NS = 16
NW = NC * NS
BPW = BATCH // NW
CH = 16
NCHUNK = BPW // CH
NV = EMBED_DIM // 16
LINPAD = VOCAB + 16

PREP_R = 1000
PREP_G = VOCAB // PREP_R


def _prep_body(emb_ref, lin_ref, c_ref):
    x = emb_ref[...]
    ss = jnp.sum(x * x, axis=1)
    c_ref[...] = lin_ref[...] - 0.5 * ss.reshape(1, 1, PREP_R)


def _fm_body(idx_hbm, emb_hbm, c_hbm, out_hbm,
             idx_v, rows0_v, rows1_v, c_v, out_v, sem0, sem1):
    wid = lax.axis_index("s") * NC + lax.axis_index("c")
    base = wid * BPW

    pltpu.sync_copy(c_hbm, c_v)
    bvec = c_v[pl.ds(VOCAB, 16)]
    lanes = lax.iota(jnp.int32, 16)
    lane0 = lanes == 0
    zeros = jnp.zeros((16,), jnp.float32)
    sems = (sem0, sem1)
    rows = (rows0_v, rows1_v)

    def fire(j, b):
        pltpu.async_copy(emb_hbm.at[idx_v.at[j]], rows[b], sems[b])

    def wait(b):
        pltpu.make_async_copy(emb_hbm.at[idx_v.at[0]], rows[b],
                              sems[b]).wait()

    def compute(j, b, ci):
        def row_acc(r, carry):
            new = list(carry)
            for v in range(NV):
                new[v] = new[v] + rows[b][r, pl.ds(v * 16, 16)]
            return tuple(new)

        accs = lax.fori_loop(0, FIELDS, row_acc, (zeros,) * NV)

        csum = zeros
        for v in range(FIELDS // 16):
            g = plsc.load_gather(c_v, [idx_v[j, pl.ds(v * 16, 16)]])
            csum = csum + g
        g = plsc.load_gather(c_v, [idx_v[j, pl.ds(FIELDS - 16, 16)]])
        csum = csum + jnp.where(lanes >= 12, g, zeros)

        t = zeros
        for v in range(NV):
            s = accs[v]
            t = t + s * s
        rvec = 0.5 * t + csum + bvec
        res = jnp.full((16,), jnp.sum(rvec), jnp.float32)
        posv = jnp.full((16,), ci * CH, jnp.int32) + j
        plsc.store_scatter(out_v, [posv], res, mask=lane0)

    for ci in range(NCHUNK):
        pltpu.sync_copy(idx_hbm.at[pl.ds(base + ci * CH, CH)], idx_v)
        fire(0, 0)

        def pair_body(p, _):
            j0 = 2 * p
            j1 = j0 + 1
            fire(j1, 1)
            wait(0)
            compute(j0, 0, ci)

            @pl.when(p < CH // 2 - 1)
            def _():
                fire(j0 + 2, 0)

            wait(1)
            compute(j1, 1, ci)
            return 0

        lax.fori_loop(0, CH // 2, pair_body, 0)

    pltpu.sync_copy(out_v, out_hbm.at[pl.ds(base, BPW)])


def kernel(interaction_pairs, emb_table, lin_table, bias):
    lin3 = lin_table.reshape((PREP_G, 1, PREP_R))
    c3 = pl.pallas_call(
        _prep_body,
        grid=(PREP_G,),
        in_specs=[
            pl.BlockSpec((PREP_R, EMBED_DIM), lambda i: (i, 0)),
            pl.BlockSpec((1, 1, PREP_R), lambda i: (i, 0, 0)),
        ],
        out_specs=pl.BlockSpec((1, 1, PREP_R), lambda i: (i, 0, 0)),
        out_shape=jax.ShapeDtypeStruct((PREP_G, 1, PREP_R), jnp.float32),
    )(emb_table, lin3)
    c_aug = jnp.concatenate(
        [c3.reshape((VOCAB,)), bias, jnp.zeros((15,), jnp.float32)])

    mesh = plsc.VectorSubcoreMesh(core_axis_name="c", subcore_axis_name="s")
    fm = pl.kernel(
        _fm_body,
        out_type=jax.ShapeDtypeStruct((BATCH,), jnp.float32),
        mesh=mesh,
        scratch_types=[
            pltpu.VMEM((CH, FIELDS), jnp.int32),
            pltpu.VMEM((FIELDS, EMBED_DIM), jnp.float32),
            pltpu.VMEM((FIELDS, EMBED_DIM), jnp.float32),
            pltpu.VMEM((LINPAD,), jnp.float32),
            pltpu.VMEM((BPW,), jnp.float32),
            pltpu.SemaphoreType.DMA,
            pltpu.SemaphoreType.DMA,
        ],
        compiler_params=pltpu.CompilerParams(needs_layout_passes=False),
    )
    return fm(interaction_pairs, emb_table, c_aug)


# speedup vs baseline: 1.2145x; 1.2145x over previous
import jax
import jax.numpy as jnp
from jax import lax
from jax.experimental import pallas as pl
from jax.experimental.pallas import tpu as pltpu
from jax.experimental.pallas import tpu_sc as plsc

BATCH = 16384
FIELDS = 100
EMBED_DIM = 128
VOCAB = 100000

NC = 2
NS = 16
NW = NC * NS
BPW = BATCH // NW
CH = 64
NCHUNK = BPW // CH
NV = EMBED_DIM // 16
HALF = FIELDS // 2
F16_SCALE = float(2.0 ** 112)


def _fm_body(idx_hbm, emb_hbm, lpk_hbm, bias_hbm, out_hbm,
             idx_v, buf0, buf1, buf2, buf3, lpk_v, bias_v, out_v,
             sem0, sem1, sem2, sem3):
    wid = lax.axis_index("s") * NC + lax.axis_index("c")
    base = wid * BPW

    pltpu.sync_copy(lpk_hbm, lpk_v)
    pltpu.sync_copy(bias_hbm, bias_v)
    bvec = bias_v[pl.ds(0, 16)]
    lanes = lax.iota(jnp.int32, 16)
    lane0 = lanes == 0
    zeros = jnp.zeros((16,), jnp.float32)
    sems = (sem0, sem1, sem2, sem3)
    bufs = (buf0, buf1, buf2, buf3)

    def fire(j, h, b):
        pltpu.async_copy(emb_hbm.at[idx_v.at[j, pl.ds(h * HALF, HALF)]],
                         bufs[b], sems[b])

    def wait(b):
        pltpu.make_async_copy(emb_hbm.at[idx_v.at[0, pl.ds(0, HALF)]],
                              bufs[b], sems[b]).wait()

    def lin_lookup(ix):
        w = plsc.load_gather(lpk_v, [ix >> 1])
        sh = w >> ((ix & 1) << 4)
        m = sh & 0x7FFF
        sign = sh & 0x8000
        return plsc.bitcast((m << 13) | (sign << 16), jnp.float32) * F16_SCALE

    def compute(j, bA, bB, ci):
        def make_acc(buf):
            def row_acc(r, carry):
                new = list(carry)
                for v in range(NV):
                    x = buf[r, pl.ds(v * 16, 16)]
                    new[v] = new[v] + x
                    new[NV + v] = new[NV + v] + x * x
                return tuple(new)
            return row_acc

        accs = lax.fori_loop(0, HALF, make_acc(bufs[bA]), (zeros,) * (2 * NV))
        accs = lax.fori_loop(0, HALF, make_acc(bufs[bB]), accs)

        lsum = zeros
        for v in range(FIELDS // 16):
            lsum = lsum + lin_lookup(idx_v[j, pl.ds(v * 16, 16)])
        g = lin_lookup(idx_v[j, pl.ds(FIELDS - 16, 16)])
        lsum = lsum + jnp.where(lanes >= 12, g, zeros)

        t = zeros
        for v in range(NV):
            s = accs[v]
            t = t + (s * s - accs[NV + v])
        rvec = 0.5 * t + lsum + bvec
        res = jnp.full((16,), jnp.sum(rvec), jnp.float32)
        posv = jnp.full((16,), ci * CH, jnp.int32) + j
        plsc.store_scatter(out_v, [posv], res, mask=lane0)

    for ci in range(NCHUNK):
        pltpu.sync_copy(idx_hbm.at[pl.ds(base + ci * CH, CH)], idx_v)
        fire(0, 0, 0)
        fire(0, 1, 1)
        fire(1, 0, 2)
        fire(1, 1, 3)

        def pair_body(p, _):
            j0 = 2 * p
            j1 = j0 + 1
            wait(0)
            wait(1)
            compute(j0, 0, 1, ci)

            @pl.when(p < CH // 2 - 1)
            def _():
                fire(j0 + 2, 0, 0)
                fire(j0 + 2, 1, 1)

            wait(2)
            wait(3)
            compute(j1, 2, 3, ci)

            @pl.when(p < CH // 2 - 1)
            def _():
                fire(j1 + 2, 0, 2)
                fire(j1 + 2, 1, 3)

            return 0

        lax.fori_loop(0, CH // 2, pair_body, 0)

    pltpu.sync_copy(out_v, out_hbm.at[pl.ds(base, BPW)])


def kernel(interaction_pairs, emb_table, lin_table, bias):
    lin_pk = lax.bitcast_convert_type(
        lin_table.astype(jnp.float16).reshape((VOCAB // 2, 2)), jnp.int32)
    bias16 = jnp.pad(bias, (0, 15))
    mesh = plsc.VectorSubcoreMesh(core_axis_name="c", subcore_axis_name="s")
    fm = pl.kernel(
        _fm_body,
        out_type=jax.ShapeDtypeStruct((BATCH,), jnp.float32),
        mesh=mesh,
        scratch_types=[
            pltpu.VMEM((CH, FIELDS), jnp.int32),
            pltpu.VMEM((HALF, EMBED_DIM), jnp.float32),
            pltpu.VMEM((HALF, EMBED_DIM), jnp.float32),
            pltpu.VMEM((HALF, EMBED_DIM), jnp.float32),
            pltpu.VMEM((HALF, EMBED_DIM), jnp.float32),
            pltpu.VMEM((VOCAB // 2,), jnp.int32),
            pltpu.VMEM((16,), jnp.float32),
            pltpu.VMEM((BPW,), jnp.float32),
            pltpu.SemaphoreType.DMA,
            pltpu.SemaphoreType.DMA,
            pltpu.SemaphoreType.DMA,
            pltpu.SemaphoreType.DMA,
        ],
        compiler_params=pltpu.CompilerParams(needs_layout_passes=False),
    )
    return fm(interaction_pairs, emb_table, lin_pk, bias16)
